# Initial kernel scaffold; baseline (speedup 1.0000x reference)
import jax
import jax.numpy as jnp
from jax.experimental import pallas as pl


def kernel(x, table, emb_bias):
    raise NotImplementedError("write your pallas kernel here")


# speedup vs baseline: 2.5068x; 2.5068x over previous
import functools

import jax
import jax.numpy as jnp
from jax import lax
from jax.experimental import pallas as pl
from jax.experimental.pallas import tpu as pltpu
from jax.experimental.pallas import tpu_sc as plsc

_B = 16384
_HIST = 50
_D = 64
_NC = 2
_NS = 16
_NW = _NC * _NS
_BAGS_PER_W = _B // _NW
_CPB = 2
_IPC = _CPB * _HIST
_CHUNKS = _BAGS_PER_W // _CPB
_NREG = _D // 16


def _sc_embedding_sum(x2d, table, emb_bias):
    mesh = plsc.VectorSubcoreMesh(
        core_axis_name="c", subcore_axis_name="s",
        num_cores=_NC, num_subcores=_NS,
    )

    @functools.partial(
        pl.kernel,
        out_type=jax.ShapeDtypeStruct((_B, _D), jnp.float32),
        mesh=mesh,
        compiler_params=pltpu.CompilerParams(use_tc_tiling_on_sc=False),
        scratch_types=[
            pltpu.VMEM((_CHUNKS, _IPC), jnp.int32),
            pltpu.VMEM((_IPC, _D), jnp.float32),
            pltpu.VMEM((_IPC, _D), jnp.float32),
            pltpu.VMEM((_BAGS_PER_W, _D), jnp.float32),
            pltpu.VMEM((_D,), jnp.float32),
            pltpu.SemaphoreType.DMA,
            pltpu.SemaphoreType.DMA,
        ],
    )
    def k(x_hbm, tab_hbm, bias_hbm, out_hbm,
          idx_v, rows0, rows1, out_v, bias_v, sem0, sem1):
        wid = lax.axis_index("s") * _NC + lax.axis_index("c")
        pltpu.sync_copy(x_hbm.at[pl.ds(wid * _CHUNKS, _CHUNKS)], idx_v)
        pltpu.sync_copy(bias_hbm, bias_v)
        bias_regs = [bias_v[pl.ds(16 * g, 16)] for g in range(_NREG)]

        def start(j, rows, sem):
            pltpu.async_copy(tab_hbm.at[idx_v.at[j]], rows, sem)

        def wait(j, rows, sem):
            pltpu.make_async_copy(tab_hbm.at[idx_v.at[j]], rows, sem).wait()

        def reduce_chunk(j, rows):
            for bag in range(_CPB):
                accs = list(bias_regs)
                for l in range(_HIST):
                    r = bag * _HIST + l
                    accs = [accs[g] + rows[r, pl.ds(16 * g, 16)]
                            for g in range(_NREG)]
                ob = j * _CPB + bag
                for g in range(_NREG):
                    out_v[ob, pl.ds(16 * g, 16)] = accs[g]

        start(0, rows0, sem0)

        def step(i, carry):
            j = 2 * i
            start(j + 1, rows1, sem1)
            wait(j, rows0, sem0)
            reduce_chunk(j, rows0)

            @pl.when(j + 2 < _CHUNKS)
            def _prefetch():
                start(j + 2, rows0, sem0)

            wait(j + 1, rows1, sem1)
            reduce_chunk(j + 1, rows1)
            return carry

        lax.fori_loop(0, _CHUNKS // 2, step, 0)
        pltpu.sync_copy(out_v, out_hbm.at[pl.ds(wid * _BAGS_PER_W, _BAGS_PER_W)])

    return k(x2d, table, emb_bias)


def kernel(x, table, emb_bias):
    x2d = x.astype(jnp.int32).reshape(_B * _HIST // _IPC, _IPC)
    return _sc_embedding_sum(x2d, table, emb_bias)
